# baseline (speedup 1.0000x reference)
import functools

import jax
import jax.numpy as jnp
from jax import lax
from jax.experimental import pallas as pl
from jax.experimental.pallas import tpu as pltpu
from jax.experimental.pallas import tpu_sc as plsc

_N_KNOTS = 32
_LANES = 16
_NC = 2
_NS = 16
_NW = _NC * _NS


def _softplus_vreg(x):
    m = jnp.maximum(x, 0.0)
    z = jnp.exp(-jnp.abs(x))
    u = 0.7 * z
    for _ in range(4):
        u = u - 1.0 + (1.0 + z) * jnp.exp(-u)
    return m + u


_N_CHUNKS = 8


def _sc_body(per_w, t_hbm, araw_hbm, out_hbm, tab_v, ptab_v, buf0, buf1,
             sin0, sin1, sout0, sout1):
    wid = lax.axis_index("s") * _NC + lax.axis_index("c")
    base = wid * per_w
    ch = per_w // _N_CHUNKS
    bufs = (buf0, buf1)
    sins = (sin0, sin1)
    souts = (sout0, sout1)

    def start_in(c):
        return pltpu.async_copy(
            t_hbm.at[pl.ds(base + c * ch, ch)], bufs[c % 2], sins[c % 2])

    def start_out(c):
        return pltpu.async_copy(
            bufs[c % 2], out_hbm.at[pl.ds(base + c * ch, ch)], souts[c % 2])

    in0 = start_in(0)
    pltpu.sync_copy(araw_hbm, tab_v.at[pl.ds(0, _N_KNOTS)])
    tab_v[pl.ds(_N_KNOTS, _LANES)] = jnp.zeros((_LANES,), jnp.float32)
    for j in range(_N_KNOTS // _LANES):
        sl = pl.ds(j * _LANES, _LANES)
        tab_v[sl] = _softplus_vreg(tab_v[sl])
    for j in range(_N_KNOTS // _LANES):
        a = tab_v[pl.ds(j * _LANES, _LANES)]
        an = tab_v[pl.ds(j * _LANES + 1, _LANES)]
        ab = plsc.bitcast(a, jnp.int32)
        db = plsc.bitcast(an - a, jnp.int32)
        hi = (ab + 0x8000) & jnp.int32(-65536)
        lo = lax.shift_right_logical(db + 0x8000, 16)
        ptab_v[pl.ds(j * _LANES, _LANES)] = hi | lo

    copies_in = [in0] + [None] * (_N_CHUNKS - 1)
    copies_out = [None] * _N_CHUNKS
    for c in range(_N_CHUNKS):
        copies_in[c].wait()
        if c >= 1:
            copies_out[c - 1].wait()
        if c + 1 < _N_CHUNKS:
            copies_in[c + 1] = start_in(c + 1)
        buf = bufs[c % 2]

        @plsc.parallel_loop(0, ch, step=_LANES, unroll=32)
        def _loop(i):
            tv = buf[pl.ds(i, _LANES)]
            i0 = tv.astype(jnp.int32)
            w = tv - i0.astype(jnp.float32)
            word = plsc.load_gather(ptab_v, [i0])
            a = plsc.bitcast(word, jnp.float32)
            d = plsc.bitcast(word << 16, jnp.float32)
            buf[pl.ds(i, _LANES)] = a + w * d

        copies_out[c] = start_out(c)
    copies_out[_N_CHUNKS - 1].wait()


def kernel(t, t_knots, alpha_raw):
    del t_knots
    tf = t.reshape(-1).astype(jnp.float32)
    n = tf.shape[0]
    per_w = n // _NW
    mesh = plsc.VectorSubcoreMesh(
        core_axis_name="c", subcore_axis_name="s",
        num_cores=_NC, num_subcores=_NS,
    )
    run = pl.kernel(
        functools.partial(_sc_body, per_w),
        out_type=jax.ShapeDtypeStruct((n,), jnp.float32),
        mesh=mesh,
        scratch_types=[
            pltpu.VMEM((_N_KNOTS + _LANES,), jnp.float32),
            pltpu.VMEM((_N_KNOTS,), jnp.int32),
            pltpu.VMEM((per_w // _N_CHUNKS,), jnp.float32),
            pltpu.VMEM((per_w // _N_CHUNKS,), jnp.float32),
            pltpu.SemaphoreType.DMA,
            pltpu.SemaphoreType.DMA,
            pltpu.SemaphoreType.DMA,
            pltpu.SemaphoreType.DMA,
        ],
        compiler_params=pltpu.CompilerParams(
            needs_layout_passes=False,
            disable_bounds_checks=True,
            disable_semaphore_checks=True,
            skip_device_barrier=True,
        ),
    )
    return run(tf, alpha_raw.astype(jnp.float32))


# speedup vs baseline: 1.4731x; 1.4731x over previous
import functools

import jax
import jax.numpy as jnp
from jax import lax
from jax.experimental import pallas as pl
from jax.experimental.pallas import tpu as pltpu
from jax.experimental.pallas import tpu_sc as plsc

_N_KNOTS = 32
_LANES = 16
_NC = 2
_NS = 16
_NW = _NC * _NS


def _softplus_vreg(x):
    m = jnp.maximum(x, 0.0)
    z = jnp.exp(-jnp.abs(x))
    u = 0.7 * z
    for _ in range(4):
        u = u - 1.0 + (1.0 + z) * jnp.exp(-u)
    return m + u


_N_CHUNKS = 8


def _sc_body(per_w, t_hbm, araw_hbm, out_hbm, tab_v, ptab_v, buf0, buf1,
             sin0, sin1, sout0, sout1):
    wid = lax.axis_index("s") * _NC + lax.axis_index("c")
    base = wid * per_w
    ch = per_w // _N_CHUNKS
    bufs = (buf0, buf1)
    sins = (sin0, sin1)
    souts = (sout0, sout1)

    def start_in(c):
        return pltpu.async_copy(
            t_hbm.at[pl.ds(base + c * ch, ch)], bufs[c % 2], sins[c % 2])

    def start_out(c):
        return pltpu.async_copy(
            bufs[c % 2], out_hbm.at[pl.ds(base + c * ch, ch)], souts[c % 2])

    in0 = start_in(0)
    pltpu.sync_copy(araw_hbm, tab_v.at[pl.ds(0, _N_KNOTS)])
    tab_v[pl.ds(_N_KNOTS, _LANES)] = jnp.zeros((_LANES,), jnp.float32)
    for j in range(_N_KNOTS // _LANES):
        sl = pl.ds(j * _LANES, _LANES)
        tab_v[sl] = _softplus_vreg(tab_v[sl])
    for j in range(_N_KNOTS // _LANES):
        a = tab_v[pl.ds(j * _LANES, _LANES)]
        an = tab_v[pl.ds(j * _LANES + 1, _LANES)]
        ab = plsc.bitcast(a, jnp.int32)
        db = plsc.bitcast(an - a, jnp.int32)
        hi = (ab + 0x8000) & jnp.int32(-65536)
        lo = lax.shift_right_logical(db + 0x8000, 16)
        ptab_v[pl.ds(j * _LANES, _LANES)] = hi | lo

    copies_in = [in0] + [None] * (_N_CHUNKS - 1)
    copies_out = [None] * _N_CHUNKS
    for c in range(_N_CHUNKS):
        copies_in[c].wait()
        if c >= 1:
            copies_out[c - 1].wait()
        if c + 1 < _N_CHUNKS:
            copies_in[c + 1] = start_in(c + 1)
        buf = bufs[c % 2]

        @plsc.parallel_loop(0, ch, step=_LANES, unroll=16)
        def _loop(i):
            tv = buf[pl.ds(i, _LANES)]
            i0 = tv.astype(jnp.int32)
            w = tv - i0.astype(jnp.float32)
            word = plsc.load_gather(ptab_v, [i0])
            a = plsc.bitcast(word, jnp.float32)
            d = plsc.bitcast(word << 16, jnp.float32)
            buf[pl.ds(i, _LANES)] = a + w * d

        copies_out[c] = start_out(c)
    copies_out[_N_CHUNKS - 1].wait()


def kernel(t, t_knots, alpha_raw):
    del t_knots
    tf = t.reshape(-1).astype(jnp.float32)
    n = tf.shape[0]
    per_w = n // _NW
    mesh = plsc.VectorSubcoreMesh(
        core_axis_name="c", subcore_axis_name="s",
        num_cores=_NC, num_subcores=_NS,
    )
    run = pl.kernel(
        functools.partial(_sc_body, per_w),
        out_type=jax.ShapeDtypeStruct((n,), jnp.float32),
        mesh=mesh,
        scratch_types=[
            pltpu.VMEM((_N_KNOTS + _LANES,), jnp.float32),
            pltpu.VMEM((_N_KNOTS,), jnp.int32),
            pltpu.VMEM((per_w // _N_CHUNKS,), jnp.float32),
            pltpu.VMEM((per_w // _N_CHUNKS,), jnp.float32),
            pltpu.SemaphoreType.DMA,
            pltpu.SemaphoreType.DMA,
            pltpu.SemaphoreType.DMA,
            pltpu.SemaphoreType.DMA,
        ],
        compiler_params=pltpu.CompilerParams(
            needs_layout_passes=False,
            disable_bounds_checks=True,
            disable_semaphore_checks=True,
            skip_device_barrier=True,
        ),
    )
    return run(tf, alpha_raw.astype(jnp.float32))
